# Initial kernel scaffold; baseline (speedup 1.0000x reference)
import jax
import jax.numpy as jnp
from jax.experimental import pallas as pl


def kernel(node_feats, edge_feats, edge_index, proj_W, proj_b, e1_W, e1_b, e2_W, e2_b, conv_b, pred_W, pred_b):
    raise NotImplementedError("write your pallas kernel here")


# speedup vs baseline: 2.8798x; 2.8798x over previous
import jax
import jax.numpy as jnp
from jax import lax
from jax.experimental import pallas as pl
from jax.experimental.pallas import tpu as pltpu
from jax.experimental.pallas import tpu_sc as plsc

N = 10000
E = 320000
D_IN = 128
D_EDGE = 16
D_OUT = 16
D_HID = 32
D_OO = D_OUT * D_OUT

NC = 2
NS = 16
NW = NC * NS
EW = E // NW
CG = 80
NCH = EW // CG
NPT = N // NS

_mesh = plsc.VectorSubcoreMesh(core_axis_name="c", subcore_axis_name="s")
_sc_params = pltpu.CompilerParams(use_tc_tiling_on_sc=False)


def _proj_body(nf_ref, w_ref, b_ref, o_ref):
    x = jnp.dot(nf_ref[...], w_ref[...], preferred_element_type=jnp.float32)
    o_ref[...] = jnp.maximum(x + b_ref[...], 0.0)


def _node_proj(nf, w, b2d):
    bm = 1000
    return pl.pallas_call(
        _proj_body,
        grid=(N // bm,),
        in_specs=[
            pl.BlockSpec((bm, D_IN), lambda i: (i, 0)),
            pl.BlockSpec((D_IN, D_OUT), lambda i: (0, 0)),
            pl.BlockSpec((1, D_OUT), lambda i: (0, 0)),
        ],
        out_specs=pl.BlockSpec((bm, D_OUT), lambda i: (i, 0)),
        out_shape=jax.ShapeDtypeStruct((N, D_OUT), jnp.float32),
    )(nf, w, b2d)


def _msg_body(ef_ref, hs_ref, w1_ref, b1_ref, w2p_ref, b2p_ref, g_ref, o_ref):
    a = jnp.dot(ef_ref[...], w1_ref[...], preferred_element_type=jnp.float32)
    a = jnp.maximum(a + b1_ref[...], 0.0)
    we = jnp.dot(a, w2p_ref[...], preferred_element_type=jnp.float32) + b2p_ref[...]
    hs = hs_ref[...]
    hst = jnp.concatenate([hs] * D_OUT, axis=1)
    o_ref[...] = jnp.dot(we * hst, g_ref[...], preferred_element_type=jnp.float32)


def _edge_messages(ef, hs, w1, b1_2d, w2p, b2p_2d, g):
    bm = 3200
    return pl.pallas_call(
        _msg_body,
        grid=(E // bm,),
        in_specs=[
            pl.BlockSpec((bm, D_EDGE), lambda i: (i, 0)),
            pl.BlockSpec((bm, D_OUT), lambda i: (i, 0)),
            pl.BlockSpec((D_EDGE, D_HID), lambda i: (0, 0)),
            pl.BlockSpec((1, D_HID), lambda i: (0, 0)),
            pl.BlockSpec((D_HID, D_OO), lambda i: (0, 0)),
            pl.BlockSpec((1, D_OO), lambda i: (0, 0)),
            pl.BlockSpec((D_OO, D_OUT), lambda i: (0, 0)),
        ],
        out_specs=pl.BlockSpec((bm, D_OUT), lambda i: (i, 0)),
        out_shape=jax.ShapeDtypeStruct((E, D_OUT), jnp.float32),
    )(ef, hs, w1, b1_2d, w2p, b2p_2d, g)


def _combine_body(p0_ref, p1_ref, b_ref, o_ref):
    o_ref[...] = jnp.maximum(p0_ref[...] + p1_ref[...] + b_ref[...], 0.0)


def _combine(p0, p1, b2d):
    bm = 1000
    return pl.pallas_call(
        _combine_body,
        grid=(N // bm,),
        in_specs=[
            pl.BlockSpec((bm, D_OUT), lambda i: (i, 0)),
            pl.BlockSpec((bm, D_OUT), lambda i: (i, 0)),
            pl.BlockSpec((1, D_OUT), lambda i: (0, 0)),
        ],
        out_specs=pl.BlockSpec((bm, D_OUT), lambda i: (i, 0)),
        out_shape=jax.ShapeDtypeStruct((N, D_OUT), jnp.float32),
    )(p0, p1, b2d)


def _final_body(p0_ref, p1_ref, b_ref, pw_ref, pb_ref, o_ref):
    h = jnp.maximum(p0_ref[...] + p1_ref[...] + b_ref[...], 0.0)
    gm = jnp.sum(h, axis=0, keepdims=True) * (1.0 / N)
    o_ref[...] = jnp.dot(gm, pw_ref[...], preferred_element_type=jnp.float32) + pb_ref[...]


def _final(p0, p1, b2d, pw, pb2d):
    return pl.pallas_call(
        _final_body,
        in_specs=[
            pl.BlockSpec((N, D_OUT), lambda: (0, 0)),
            pl.BlockSpec((N, D_OUT), lambda: (0, 0)),
            pl.BlockSpec((1, D_OUT), lambda: (0, 0)),
            pl.BlockSpec((D_OUT, 2), lambda: (0, 0)),
            pl.BlockSpec((1, 2), lambda: (0, 0)),
        ],
        out_specs=pl.BlockSpec((1, 2), lambda: (0, 0)),
        out_shape=jax.ShapeDtypeStruct((1, 2), jnp.float32),
    )(p0, p1, b2d, pw, pb2d)


def _gather_body(h_hbm, src_hbm, out_hbm, idx_v, rows_v, sem):
    cid = lax.axis_index("c")
    sid = lax.axis_index("s")
    w = cid * NS + sid
    pltpu.sync_copy(src_hbm.at[w], idx_v)
    base = w * EW

    def body(j, carry):
        pltpu.async_copy(h_hbm.at[idx_v.at[j]], rows_v, sem).wait()
        pltpu.sync_copy(rows_v, out_hbm.at[pl.ds(base + j * CG, CG)])
        return carry

    lax.fori_loop(0, NCH, body, 0)


def _gather(h, src3):
    return pl.kernel(
        _gather_body,
        out_type=jax.ShapeDtypeStruct((E, D_OUT), jnp.float32),
        mesh=_mesh,
        scratch_types=[
            pltpu.VMEM((NCH, CG), jnp.int32),
            pltpu.VMEM((CG, D_OUT), jnp.float32),
            pltpu.SemaphoreType.DMA,
        ],
        compiler_params=_sc_params,
    )(h, src3)


def _scatter_body(m_hbm, dst_hbm, z_hbm, out_hbm, agg_sh, idx_v, m_v, row_v):
    cid = lax.axis_index("c")
    sid = lax.axis_index("s")
    w = cid * NS + sid
    pltpu.sync_copy(z_hbm.at[pl.ds(sid * NPT, NPT)], row_v)
    pltpu.sync_copy(row_v, agg_sh.at[pl.ds(sid * NPT, NPT)])
    pltpu.sync_copy(dst_hbm.at[w], idx_v)
    plsc.subcore_barrier()
    base = w * EW

    def body(j, carry):
        pltpu.sync_copy(m_hbm.at[pl.ds(base + j * CG, CG)], m_v)
        pltpu.sync_copy(m_v, agg_sh.at[idx_v.at[j]], add=True)
        return carry

    lax.fori_loop(0, NCH, body, 0)
    plsc.subcore_barrier()
    pltpu.sync_copy(agg_sh.at[pl.ds(sid * NPT, NPT)], row_v)
    pltpu.sync_copy(row_v, out_hbm.at[cid].at[pl.ds(sid * NPT, NPT)])


def _scatter(m, dst3, zeros):
    return pl.kernel(
        _scatter_body,
        out_type=jax.ShapeDtypeStruct((NC, N, D_OUT), jnp.float32),
        mesh=_mesh,
        scratch_types=[
            pltpu.VMEM_SHARED((N, D_OUT), jnp.float32),
            pltpu.VMEM((NCH, CG), jnp.int32),
            pltpu.VMEM((CG, D_OUT), jnp.float32),
            pltpu.VMEM((NPT, D_OUT), jnp.float32),
        ],
        compiler_params=_sc_params,
    )(m, dst3, zeros)


def kernel(node_feats, edge_feats, edge_index, proj_W, proj_b, e1_W, e1_b,
           e2_W, e2_b, conv_b, pred_W, pred_b):
    src3 = edge_index[0].astype(jnp.int32).reshape(NW, NCH, CG)
    dst3 = edge_index[1].astype(jnp.int32).reshape(NW, NCH, CG)

    j = jnp.arange(D_OO)
    perm = (j % D_OUT) * D_OUT + j // D_OUT
    w2p = e2_W[:, perm]
    b2p = e2_b[perm].reshape(1, D_OO)
    g = jnp.repeat(jnp.eye(D_OUT, dtype=jnp.float32), D_OUT, axis=0)
    zeros = jnp.zeros((N, D_OUT), jnp.float32)

    h = _node_proj(node_feats, proj_W, proj_b.reshape(1, D_OUT))
    parts = None
    for step in range(2):
        hs = _gather(h, src3)
        m = _edge_messages(edge_feats, hs, e1_W, e1_b.reshape(1, D_HID),
                           w2p, b2p, g)
        parts = _scatter(m, dst3, zeros)
        if step == 0:
            h = _combine(parts[0], parts[1], conv_b.reshape(1, D_OUT))
    return _final(parts[0], parts[1], conv_b.reshape(1, D_OUT),
                  pred_W, pred_b.reshape(1, 2))


# speedup vs baseline: 3.5156x; 1.2208x over previous
import jax
import jax.numpy as jnp
from jax import lax
from jax.experimental import pallas as pl
from jax.experimental.pallas import tpu as pltpu
from jax.experimental.pallas import tpu_sc as plsc

N = 10000
E = 320000
D_IN = 128
D_EDGE = 16
D_OUT = 16
D_HID = 32
D_OO = D_OUT * D_OUT

NC = 2
NS = 16
NW = NC * NS
EW = E // NW
CG = 80
NCH = EW // CG
NPT = N // NS
SUP = 25
NSUP = NCH // SUP
SB = SUP * CG

_mesh = plsc.VectorSubcoreMesh(core_axis_name="c", subcore_axis_name="s")
_sc_params = pltpu.CompilerParams(use_tc_tiling_on_sc=False)


def _proj_body(nf_ref, w_ref, b_ref, o_ref):
    x = jnp.dot(nf_ref[...], w_ref[...], preferred_element_type=jnp.float32)
    o_ref[...] = jnp.maximum(x + b_ref[...], 0.0)


def _node_proj(nf, w, b2d):
    bm = 1000
    return pl.pallas_call(
        _proj_body,
        grid=(N // bm,),
        in_specs=[
            pl.BlockSpec((bm, D_IN), lambda i: (i, 0)),
            pl.BlockSpec((D_IN, D_OUT), lambda i: (0, 0)),
            pl.BlockSpec((1, D_OUT), lambda i: (0, 0)),
        ],
        out_specs=pl.BlockSpec((bm, D_OUT), lambda i: (i, 0)),
        out_shape=jax.ShapeDtypeStruct((N, D_OUT), jnp.float32),
    )(nf, w, b2d)


def _msg_body(ef_ref, hs_ref, w1_ref, b1_ref, w2p_ref, b2p_ref, g_ref, o_ref):
    a = jnp.dot(ef_ref[...], w1_ref[...], preferred_element_type=jnp.float32)
    a = jnp.maximum(a + b1_ref[...], 0.0).astype(jnp.bfloat16)
    we = jnp.dot(a, w2p_ref[...], preferred_element_type=jnp.float32) + b2p_ref[...]
    hs = hs_ref[...]
    hst = jnp.concatenate([hs] * D_OUT, axis=1)
    o_ref[...] = jnp.dot(we * hst, g_ref[...], preferred_element_type=jnp.float32)


def _edge_messages(ef, hs, w1, b1_2d, w2p, b2p_2d, g):
    bm = 3200
    return pl.pallas_call(
        _msg_body,
        grid=(E // bm,),
        in_specs=[
            pl.BlockSpec((bm, D_EDGE), lambda i: (i, 0)),
            pl.BlockSpec((bm, D_OUT), lambda i: (i, 0)),
            pl.BlockSpec((D_EDGE, D_HID), lambda i: (0, 0)),
            pl.BlockSpec((1, D_HID), lambda i: (0, 0)),
            pl.BlockSpec((D_HID, D_OO), lambda i: (0, 0)),
            pl.BlockSpec((1, D_OO), lambda i: (0, 0)),
            pl.BlockSpec((D_OO, D_OUT), lambda i: (0, 0)),
        ],
        out_specs=pl.BlockSpec((bm, D_OUT), lambda i: (i, 0)),
        out_shape=jax.ShapeDtypeStruct((E, D_OUT), jnp.float32),
    )(ef, hs, w1, b1_2d, w2p, b2p_2d, g)


def _combine_body(p0_ref, p1_ref, b_ref, o_ref):
    o_ref[...] = jnp.maximum(p0_ref[...] + p1_ref[...] + b_ref[...], 0.0)


def _combine(p0, p1, b2d):
    bm = 1000
    return pl.pallas_call(
        _combine_body,
        grid=(N // bm,),
        in_specs=[
            pl.BlockSpec((bm, D_OUT), lambda i: (i, 0)),
            pl.BlockSpec((bm, D_OUT), lambda i: (i, 0)),
            pl.BlockSpec((1, D_OUT), lambda i: (0, 0)),
        ],
        out_specs=pl.BlockSpec((bm, D_OUT), lambda i: (i, 0)),
        out_shape=jax.ShapeDtypeStruct((N, D_OUT), jnp.float32),
    )(p0, p1, b2d)


def _final_body(p0_ref, p1_ref, b_ref, pw_ref, pb_ref, o_ref):
    h = jnp.maximum(p0_ref[...] + p1_ref[...] + b_ref[...], 0.0)
    gm = jnp.sum(h, axis=0, keepdims=True) * (1.0 / N)
    o_ref[...] = jnp.dot(gm, pw_ref[...], preferred_element_type=jnp.float32) + pb_ref[...]


def _final(p0, p1, b2d, pw, pb2d):
    return pl.pallas_call(
        _final_body,
        in_specs=[
            pl.BlockSpec((N, D_OUT), lambda: (0, 0)),
            pl.BlockSpec((N, D_OUT), lambda: (0, 0)),
            pl.BlockSpec((1, D_OUT), lambda: (0, 0)),
            pl.BlockSpec((D_OUT, 2), lambda: (0, 0)),
            pl.BlockSpec((1, 2), lambda: (0, 0)),
        ],
        out_specs=pl.BlockSpec((1, 2), lambda: (0, 0)),
        out_shape=jax.ShapeDtypeStruct((1, 2), jnp.float32),
    )(p0, p1, b2d, pw, pb2d)


def _gather_body(h_hbm, src_hbm, out_hbm, idx_v, rows_v, sem):
    cid = lax.axis_index("c")
    sid = lax.axis_index("s")
    w = cid * NS + sid
    pltpu.sync_copy(src_hbm.at[w], idx_v)
    base = w * EW

    def sup(s, carry):
        descs = [
            pltpu.async_copy(
                h_hbm.at[idx_v.at[s * SUP + c]],
                rows_v.at[pl.ds(c * CG, CG)], sem)
            for c in range(SUP)
        ]
        for d in descs:
            d.wait()
        pltpu.sync_copy(rows_v, out_hbm.at[pl.ds(base + s * SB, SB)])
        return carry

    lax.fori_loop(0, NSUP, sup, 0)


def _gather(h, src3):
    return pl.kernel(
        _gather_body,
        out_type=jax.ShapeDtypeStruct((E, D_OUT), jnp.float32),
        mesh=_mesh,
        scratch_types=[
            pltpu.VMEM((NCH, CG), jnp.int32),
            pltpu.VMEM((SB, D_OUT), jnp.float32),
            pltpu.SemaphoreType.DMA,
        ],
        compiler_params=_sc_params,
    )(h, src3)


def _scatter_body(m_hbm, dst_hbm, z_hbm, out_hbm, agg_sh, idx_v, m_v, row_v, sem):
    cid = lax.axis_index("c")
    sid = lax.axis_index("s")
    w = cid * NS + sid
    pltpu.sync_copy(z_hbm.at[pl.ds(sid * NPT, NPT)], row_v)
    pltpu.sync_copy(row_v, agg_sh.at[pl.ds(sid * NPT, NPT)])
    pltpu.sync_copy(dst_hbm.at[w], idx_v)
    plsc.subcore_barrier()
    base = w * EW

    def sup(s, carry):
        pltpu.sync_copy(m_hbm.at[pl.ds(base + s * SB, SB)], m_v)
        descs = [
            pltpu.async_copy(
                m_v.at[pl.ds(c * CG, CG)],
                agg_sh.at[idx_v.at[s * SUP + c]], sem, add=True)
            for c in range(SUP)
        ]
        for d in descs:
            d.wait()
        return carry

    lax.fori_loop(0, NSUP, sup, 0)
    plsc.subcore_barrier()
    pltpu.sync_copy(agg_sh.at[pl.ds(sid * NPT, NPT)], row_v)
    pltpu.sync_copy(row_v, out_hbm.at[cid].at[pl.ds(sid * NPT, NPT)])


def _scatter(m, dst3, zeros):
    return pl.kernel(
        _scatter_body,
        out_type=jax.ShapeDtypeStruct((NC, N, D_OUT), jnp.float32),
        mesh=_mesh,
        scratch_types=[
            pltpu.VMEM_SHARED((N, D_OUT), jnp.float32),
            pltpu.VMEM((NCH, CG), jnp.int32),
            pltpu.VMEM((SB, D_OUT), jnp.float32),
            pltpu.VMEM((NPT, D_OUT), jnp.float32),
            pltpu.SemaphoreType.DMA,
        ],
        compiler_params=_sc_params,
    )(m, dst3, zeros)


def kernel(node_feats, edge_feats, edge_index, proj_W, proj_b, e1_W, e1_b,
           e2_W, e2_b, conv_b, pred_W, pred_b):
    src3 = edge_index[0].astype(jnp.int32).reshape(NW, NCH, CG)
    dst3 = edge_index[1].astype(jnp.int32).reshape(NW, NCH, CG)

    j = jnp.arange(D_OO)
    perm = (j % D_OUT) * D_OUT + j // D_OUT
    w2p = e2_W[:, perm]
    b2p = e2_b[perm].reshape(1, D_OO)
    g = jnp.repeat(jnp.eye(D_OUT, dtype=jnp.float32), D_OUT, axis=0)
    zeros = jnp.zeros((N, D_OUT), jnp.float32)

    ef16 = edge_feats.astype(jnp.bfloat16)
    w1_16 = e1_W.astype(jnp.bfloat16)
    w2p16 = w2p.astype(jnp.bfloat16)

    h = _node_proj(node_feats, proj_W, proj_b.reshape(1, D_OUT))
    parts = None
    for step in range(2):
        hs = _gather(h, src3)
        m = _edge_messages(ef16, hs, w1_16, e1_b.reshape(1, D_HID),
                           w2p16, b2p, g)
        parts = _scatter(m, dst3, zeros)
        if step == 0:
            h = _combine(parts[0], parts[1], conv_b.reshape(1, D_OUT))
    return _final(parts[0], parts[1], conv_b.reshape(1, D_OUT),
                  pred_W, pred_b.reshape(1, 2))


# speedup vs baseline: 6.4624x; 1.8382x over previous
import jax
import jax.numpy as jnp
from jax import lax
from jax.experimental import pallas as pl
from jax.experimental.pallas import tpu as pltpu
from jax.experimental.pallas import tpu_sc as plsc

N = 10000
E = 320000
D_IN = 128
D_EDGE = 16
D_OUT = 16
D_HID = 32
D_OO = D_OUT * D_OUT

NC = 2
NS = 16
NW = NC * NS
EW = E // NW
CG = 80
NCH = EW // CG
NPT = N // NS
SUP = 25
NSUP = NCH // SUP
SB = SUP * CG

_mesh = plsc.VectorSubcoreMesh(core_axis_name="c", subcore_axis_name="s")
_sc_params = pltpu.CompilerParams(use_tc_tiling_on_sc=False)


def _proj_body(nf_ref, w_ref, b_ref, o_ref):
    x = jnp.dot(nf_ref[...], w_ref[...], preferred_element_type=jnp.float32)
    o_ref[...] = jnp.maximum(x + b_ref[...], 0.0)


def _node_proj(nf, w, b2d):
    bm = 1000
    return pl.pallas_call(
        _proj_body,
        grid=(N // bm,),
        in_specs=[
            pl.BlockSpec((bm, D_IN), lambda i: (i, 0)),
            pl.BlockSpec((D_IN, D_OUT), lambda i: (0, 0)),
            pl.BlockSpec((1, D_OUT), lambda i: (0, 0)),
        ],
        out_specs=pl.BlockSpec((bm, D_OUT), lambda i: (i, 0)),
        out_shape=jax.ShapeDtypeStruct((N, D_OUT), jnp.float32),
    )(nf, w, b2d)


def _msg_body(ef_ref, hp_ref, w1_ref, b1_ref, w2p_ref, b2p_ref, q_ref, g_ref,
              o_ref):
    a = jnp.dot(ef_ref[...], w1_ref[...], preferred_element_type=jnp.float32)
    a = jnp.maximum(a + b1_ref[...], 0.0).astype(jnp.bfloat16)
    we = jnp.dot(a, w2p_ref[...], preferred_element_type=jnp.float32)
    we = we + b2p_ref[...]
    hpt = jnp.dot(hp_ref[...].astype(jnp.bfloat16), q_ref[...],
                  preferred_element_type=jnp.float32)
    prod = (we * hpt).astype(jnp.bfloat16)
    o_ref[...] = jnp.dot(prod, g_ref[...],
                         preferred_element_type=jnp.float32)


def _edge_messages(ef_p, hp, w1_p, b1_p, w2_p, b2_p, q_p, g_p):
    bm = 400
    ep8 = E // 8
    return pl.pallas_call(
        _msg_body,
        grid=(ep8 // bm,),
        in_specs=[
            pl.BlockSpec((bm, 128), lambda i: (i, 0)),
            pl.BlockSpec((bm, 128), lambda i: (i, 0)),
            pl.BlockSpec((128, 8 * D_HID), lambda i: (0, 0)),
            pl.BlockSpec((1, 8 * D_HID), lambda i: (0, 0)),
            pl.BlockSpec((8 * D_HID, 8 * D_OO), lambda i: (0, 0)),
            pl.BlockSpec((1, 8 * D_OO), lambda i: (0, 0)),
            pl.BlockSpec((128, 8 * D_OO), lambda i: (0, 0)),
            pl.BlockSpec((8 * D_OO, 128), lambda i: (0, 0)),
        ],
        out_specs=pl.BlockSpec((bm, 128), lambda i: (i, 0)),
        out_shape=jax.ShapeDtypeStruct((ep8, 128), jnp.float32),
    )(ef_p, hp, w1_p, b1_p, w2_p, b2_p, q_p, g_p)


def _combine_body(p0_ref, p1_ref, b_ref, o_ref):
    o_ref[...] = jnp.maximum(p0_ref[...] + p1_ref[...] + b_ref[...], 0.0)


def _combine(p0, p1, b2d):
    bm = 1000
    return pl.pallas_call(
        _combine_body,
        grid=(N // bm,),
        in_specs=[
            pl.BlockSpec((bm, D_OUT), lambda i: (i, 0)),
            pl.BlockSpec((bm, D_OUT), lambda i: (i, 0)),
            pl.BlockSpec((1, D_OUT), lambda i: (0, 0)),
        ],
        out_specs=pl.BlockSpec((bm, D_OUT), lambda i: (i, 0)),
        out_shape=jax.ShapeDtypeStruct((N, D_OUT), jnp.float32),
    )(p0, p1, b2d)


def _final_body(p0_ref, p1_ref, b_ref, pw_ref, pb_ref, o_ref):
    h = jnp.maximum(p0_ref[...] + p1_ref[...] + b_ref[...], 0.0)
    gm = jnp.sum(h, axis=0, keepdims=True) * (1.0 / N)
    o_ref[...] = jnp.dot(gm, pw_ref[...], preferred_element_type=jnp.float32) + pb_ref[...]


def _final(p0, p1, b2d, pw, pb2d):
    return pl.pallas_call(
        _final_body,
        in_specs=[
            pl.BlockSpec((N, D_OUT), lambda: (0, 0)),
            pl.BlockSpec((N, D_OUT), lambda: (0, 0)),
            pl.BlockSpec((1, D_OUT), lambda: (0, 0)),
            pl.BlockSpec((D_OUT, 2), lambda: (0, 0)),
            pl.BlockSpec((1, 2), lambda: (0, 0)),
        ],
        out_specs=pl.BlockSpec((1, 2), lambda: (0, 0)),
        out_shape=jax.ShapeDtypeStruct((1, 2), jnp.float32),
    )(p0, p1, b2d, pw, pb2d)


def _gather_body(h_hbm, src_hbm, out_hbm, idx_v, rows_v, sem):
    cid = lax.axis_index("c")
    sid = lax.axis_index("s")
    w = cid * NS + sid
    pltpu.sync_copy(src_hbm.at[w], idx_v)
    base = w * EW

    def sup(s, carry):
        descs = [
            pltpu.async_copy(
                h_hbm.at[idx_v.at[s * SUP + c]],
                rows_v.at[pl.ds(c * CG, CG)], sem)
            for c in range(SUP)
        ]
        for d in descs:
            d.wait()
        pltpu.sync_copy(rows_v, out_hbm.at[pl.ds(base + s * SB, SB)])
        return carry

    lax.fori_loop(0, NSUP, sup, 0)


def _gather(h, src3):
    return pl.kernel(
        _gather_body,
        out_type=jax.ShapeDtypeStruct((E, D_OUT), jnp.float32),
        mesh=_mesh,
        scratch_types=[
            pltpu.VMEM((NCH, CG), jnp.int32),
            pltpu.VMEM((SB, D_OUT), jnp.float32),
            pltpu.SemaphoreType.DMA,
        ],
        compiler_params=_sc_params,
    )(h, src3)


def _scatter_body(m_hbm, dst_hbm, z_hbm, out_hbm, agg_sh, idx_v, m_v, row_v, sem):
    cid = lax.axis_index("c")
    sid = lax.axis_index("s")
    w = cid * NS + sid
    pltpu.sync_copy(z_hbm.at[pl.ds(sid * NPT, NPT)], row_v)
    pltpu.sync_copy(row_v, agg_sh.at[pl.ds(sid * NPT, NPT)])
    pltpu.sync_copy(dst_hbm.at[w], idx_v)
    plsc.subcore_barrier()
    base = w * EW

    def sup(s, carry):
        pltpu.sync_copy(m_hbm.at[pl.ds(base + s * SB, SB)], m_v)
        descs = [
            pltpu.async_copy(
                m_v.at[pl.ds(c * CG, CG)],
                agg_sh.at[idx_v.at[s * SUP + c]], sem, add=True)
            for c in range(SUP)
        ]
        for d in descs:
            d.wait()
        return carry

    lax.fori_loop(0, NSUP, sup, 0)
    plsc.subcore_barrier()
    pltpu.sync_copy(agg_sh.at[pl.ds(sid * NPT, NPT)], row_v)
    pltpu.sync_copy(row_v, out_hbm.at[cid].at[pl.ds(sid * NPT, NPT)])


def _scatter(m, dst3, zeros):
    return pl.kernel(
        _scatter_body,
        out_type=jax.ShapeDtypeStruct((NC, N, D_OUT), jnp.float32),
        mesh=_mesh,
        scratch_types=[
            pltpu.VMEM_SHARED((N, D_OUT), jnp.float32),
            pltpu.VMEM((NCH, CG), jnp.int32),
            pltpu.VMEM((SB, D_OUT), jnp.float32),
            pltpu.VMEM((NPT, D_OUT), jnp.float32),
            pltpu.SemaphoreType.DMA,
        ],
        compiler_params=_sc_params,
    )(m, dst3, zeros)


def kernel(node_feats, edge_feats, edge_index, proj_W, proj_b, e1_W, e1_b,
           e2_W, e2_b, conv_b, pred_W, pred_b):
    src3 = edge_index[0].astype(jnp.int32).reshape(NW, NCH, CG)
    dst3 = edge_index[1].astype(jnp.int32).reshape(NW, NCH, CG)

    j = jnp.arange(D_OO)
    perm = (j % D_OUT) * D_OUT + j // D_OUT
    w2perm = e2_W[:, perm]
    b2perm = e2_b[perm]
    eye8 = jnp.eye(8, dtype=jnp.float32)
    w1_p = jnp.kron(eye8, e1_W).astype(jnp.bfloat16)
    b1_p = jnp.tile(e1_b, 8).reshape(1, 8 * D_HID)
    w2_p = jnp.kron(eye8, w2perm).astype(jnp.bfloat16)
    b2_p = jnp.tile(b2perm, 8).reshape(1, 8 * D_OO)
    q16 = jnp.concatenate([jnp.eye(D_OUT, dtype=jnp.float32)] * D_OUT, axis=1)
    q_p = jnp.kron(eye8, q16).astype(jnp.bfloat16)
    g16 = jnp.repeat(jnp.eye(D_OUT, dtype=jnp.float32), D_OUT, axis=0)
    g_p = jnp.kron(eye8, g16).astype(jnp.bfloat16)
    zeros = jnp.zeros((N, D_OUT), jnp.float32)
    ef_p = edge_feats.reshape(E // 8, 128).astype(jnp.bfloat16)

    h = _node_proj(node_feats, proj_W, proj_b.reshape(1, D_OUT))
    parts = None
    for step in range(2):
        hp = _gather(h, src3).reshape(E // 8, 128)
        m = _edge_messages(ef_p, hp, w1_p, b1_p, w2_p, b2_p, q_p, g_p)
        parts = _scatter(m.reshape(E, D_OUT), dst3, zeros)
        if step == 0:
            h = _combine(parts[0], parts[1], conv_b.reshape(1, D_OUT))
    return _final(parts[0], parts[1], conv_b.reshape(1, D_OUT),
                  pred_W, pred_b.reshape(1, 2))


# speedup vs baseline: 6.5137x; 1.0079x over previous
import jax
import jax.numpy as jnp
from jax import lax
from jax.experimental import pallas as pl
from jax.experimental.pallas import tpu as pltpu
from jax.experimental.pallas import tpu_sc as plsc

N = 10000
E = 320000
D_IN = 128
D_EDGE = 16
D_OUT = 16
D_HID = 32
D_OO = D_OUT * D_OUT

NC = 2
NS = 16
NW = NC * NS
EW = E // NW
CG = 80
NCH = EW // CG
NPT = N // NS
SUP = 25
NSUP = NCH // SUP
SB = SUP * CG

_mesh = plsc.VectorSubcoreMesh(core_axis_name="c", subcore_axis_name="s")
_sc_params = pltpu.CompilerParams(use_tc_tiling_on_sc=False)


def _proj_body(nf_ref, w_ref, b_ref, o_ref):
    x = jnp.dot(nf_ref[...], w_ref[...], preferred_element_type=jnp.float32)
    o_ref[...] = jnp.maximum(x + b_ref[...], 0.0)


def _node_proj(nf, w, b2d):
    bm = 1000
    return pl.pallas_call(
        _proj_body,
        grid=(N // bm,),
        in_specs=[
            pl.BlockSpec((bm, D_IN), lambda i: (i, 0)),
            pl.BlockSpec((D_IN, D_OUT), lambda i: (0, 0)),
            pl.BlockSpec((1, D_OUT), lambda i: (0, 0)),
        ],
        out_specs=pl.BlockSpec((bm, D_OUT), lambda i: (i, 0)),
        out_shape=jax.ShapeDtypeStruct((N, D_OUT), jnp.float32),
    )(nf, w, b2d)


def _we_body(ef_ref, w1_ref, b1_ref, w2p_ref, b2p_ref, o_ref):
    a = jnp.dot(ef_ref[...].astype(jnp.bfloat16), w1_ref[...],
                preferred_element_type=jnp.float32)
    a = jnp.maximum(a + b1_ref[...], 0.0).astype(jnp.bfloat16)
    we = jnp.dot(a, w2p_ref[...], preferred_element_type=jnp.float32)
    o_ref[...] = (we + b2p_ref[...]).astype(jnp.bfloat16)


def _edge_weights(ef_p, w1_p, b1_p, w2_p, b2_p):
    bm = 400
    ep8 = E // 8
    return pl.pallas_call(
        _we_body,
        grid=(ep8 // bm,),
        in_specs=[
            pl.BlockSpec((bm, 128), lambda i: (i, 0)),
            pl.BlockSpec((128, 8 * D_HID), lambda i: (0, 0)),
            pl.BlockSpec((1, 8 * D_HID), lambda i: (0, 0)),
            pl.BlockSpec((8 * D_HID, 8 * D_OO), lambda i: (0, 0)),
            pl.BlockSpec((1, 8 * D_OO), lambda i: (0, 0)),
        ],
        out_specs=pl.BlockSpec((bm, 8 * D_OO), lambda i: (i, 0)),
        out_shape=jax.ShapeDtypeStruct((ep8, 8 * D_OO), jnp.bfloat16),
    )(ef_p, w1_p, b1_p, w2_p, b2_p)


def _msg_body(we_ref, hp_ref, q_ref, g_ref, o_ref):
    hpt = jnp.dot(hp_ref[...].astype(jnp.bfloat16), q_ref[...],
                  preferred_element_type=jnp.float32).astype(jnp.bfloat16)
    prod = we_ref[...] * hpt
    o_ref[...] = jnp.dot(prod, g_ref[...],
                         preferred_element_type=jnp.float32)


def _edge_messages(we_p, hp, q_p, g_p):
    bm = 400
    ep8 = E // 8
    return pl.pallas_call(
        _msg_body,
        grid=(ep8 // bm,),
        in_specs=[
            pl.BlockSpec((bm, 8 * D_OO), lambda i: (i, 0)),
            pl.BlockSpec((bm, 128), lambda i: (i, 0)),
            pl.BlockSpec((128, 8 * D_OO), lambda i: (0, 0)),
            pl.BlockSpec((8 * D_OO, 128), lambda i: (0, 0)),
        ],
        out_specs=pl.BlockSpec((bm, 128), lambda i: (i, 0)),
        out_shape=jax.ShapeDtypeStruct((ep8, 128), jnp.float32),
    )(we_p, hp, q_p, g_p)


def _combine_body(p0_ref, p1_ref, b_ref, o_ref):
    o_ref[...] = jnp.maximum(p0_ref[...] + p1_ref[...] + b_ref[...], 0.0)


def _combine(p0, p1, b2d):
    bm = 1000
    return pl.pallas_call(
        _combine_body,
        grid=(N // bm,),
        in_specs=[
            pl.BlockSpec((bm, D_OUT), lambda i: (i, 0)),
            pl.BlockSpec((bm, D_OUT), lambda i: (i, 0)),
            pl.BlockSpec((1, D_OUT), lambda i: (0, 0)),
        ],
        out_specs=pl.BlockSpec((bm, D_OUT), lambda i: (i, 0)),
        out_shape=jax.ShapeDtypeStruct((N, D_OUT), jnp.float32),
    )(p0, p1, b2d)


def _final_body(p0_ref, p1_ref, b_ref, pw_ref, pb_ref, o_ref):
    h = jnp.maximum(p0_ref[...] + p1_ref[...] + b_ref[...], 0.0)
    gm = jnp.sum(h, axis=0, keepdims=True) * (1.0 / N)
    o_ref[...] = jnp.dot(gm, pw_ref[...], preferred_element_type=jnp.float32) + pb_ref[...]


def _final(p0, p1, b2d, pw, pb2d):
    return pl.pallas_call(
        _final_body,
        in_specs=[
            pl.BlockSpec((N, D_OUT), lambda: (0, 0)),
            pl.BlockSpec((N, D_OUT), lambda: (0, 0)),
            pl.BlockSpec((1, D_OUT), lambda: (0, 0)),
            pl.BlockSpec((D_OUT, 2), lambda: (0, 0)),
            pl.BlockSpec((1, 2), lambda: (0, 0)),
        ],
        out_specs=pl.BlockSpec((1, 2), lambda: (0, 0)),
        out_shape=jax.ShapeDtypeStruct((1, 2), jnp.float32),
    )(p0, p1, b2d, pw, pb2d)


def _gather_body(h_hbm, src_hbm, out_hbm, idx_v, rows_v, sem):
    cid = lax.axis_index("c")
    sid = lax.axis_index("s")
    w = cid * NS + sid
    pltpu.sync_copy(src_hbm.at[w], idx_v)
    base = w * EW

    def sup(s, carry):
        descs = [
            pltpu.async_copy(
                h_hbm.at[idx_v.at[s * SUP + c]],
                rows_v.at[pl.ds(c * CG, CG)], sem)
            for c in range(SUP)
        ]
        for d in descs:
            d.wait()
        pltpu.sync_copy(rows_v, out_hbm.at[pl.ds(base + s * SB, SB)])
        return carry

    lax.fori_loop(0, NSUP, sup, 0)


def _gather(h, src3):
    return pl.kernel(
        _gather_body,
        out_type=jax.ShapeDtypeStruct((E, D_OUT), jnp.float32),
        mesh=_mesh,
        scratch_types=[
            pltpu.VMEM((NCH, CG), jnp.int32),
            pltpu.VMEM((SB, D_OUT), jnp.float32),
            pltpu.SemaphoreType.DMA,
        ],
        compiler_params=_sc_params,
    )(h, src3)


def _scatter_body(m_hbm, dst_hbm, z_hbm, out_hbm, agg_sh, idx_v, m_v, row_v, sem):
    cid = lax.axis_index("c")
    sid = lax.axis_index("s")
    w = cid * NS + sid
    pltpu.sync_copy(z_hbm.at[pl.ds(sid * NPT, NPT)], row_v)
    pltpu.sync_copy(row_v, agg_sh.at[pl.ds(sid * NPT, NPT)])
    pltpu.sync_copy(dst_hbm.at[w], idx_v)
    plsc.subcore_barrier()
    base = w * EW

    def sup(s, carry):
        pltpu.sync_copy(m_hbm.at[pl.ds(base + s * SB, SB)], m_v)
        descs = [
            pltpu.async_copy(
                m_v.at[pl.ds(c * CG, CG)],
                agg_sh.at[idx_v.at[s * SUP + c]], sem, add=True)
            for c in range(SUP)
        ]
        for d in descs:
            d.wait()
        return carry

    lax.fori_loop(0, NSUP, sup, 0)
    plsc.subcore_barrier()
    pltpu.sync_copy(agg_sh.at[pl.ds(sid * NPT, NPT)], row_v)
    pltpu.sync_copy(row_v, out_hbm.at[cid].at[pl.ds(sid * NPT, NPT)])


def _scatter(m, dst3, zeros):
    return pl.kernel(
        _scatter_body,
        out_type=jax.ShapeDtypeStruct((NC, N, D_OUT), jnp.float32),
        mesh=_mesh,
        scratch_types=[
            pltpu.VMEM_SHARED((N, D_OUT), jnp.float32),
            pltpu.VMEM((NCH, CG), jnp.int32),
            pltpu.VMEM((SB, D_OUT), jnp.float32),
            pltpu.VMEM((NPT, D_OUT), jnp.float32),
            pltpu.SemaphoreType.DMA,
        ],
        compiler_params=_sc_params,
    )(m, dst3, zeros)


def kernel(node_feats, edge_feats, edge_index, proj_W, proj_b, e1_W, e1_b,
           e2_W, e2_b, conv_b, pred_W, pred_b):
    src3 = edge_index[0].astype(jnp.int32).reshape(NW, NCH, CG)
    dst3 = edge_index[1].astype(jnp.int32).reshape(NW, NCH, CG)

    j = jnp.arange(D_OO)
    perm = (j % D_OUT) * D_OUT + j // D_OUT
    w2perm = e2_W[:, perm]
    b2perm = e2_b[perm]
    eye8 = jnp.eye(8, dtype=jnp.float32)
    w1_p = jnp.kron(eye8, e1_W).astype(jnp.bfloat16)
    b1_p = jnp.tile(e1_b, 8).reshape(1, 8 * D_HID)
    w2_p = jnp.kron(eye8, w2perm).astype(jnp.bfloat16)
    b2_p = jnp.tile(b2perm, 8).reshape(1, 8 * D_OO)
    q16 = jnp.concatenate([jnp.eye(D_OUT, dtype=jnp.float32)] * D_OUT, axis=1)
    q_p = jnp.kron(eye8, q16).astype(jnp.bfloat16)
    g16 = jnp.repeat(jnp.eye(D_OUT, dtype=jnp.float32), D_OUT, axis=0)
    g_p = jnp.kron(eye8, g16).astype(jnp.bfloat16)
    zeros = jnp.zeros((N, D_OUT), jnp.float32)
    ef_p = edge_feats.reshape(E // 8, 128)

    we_p = _edge_weights(ef_p, w1_p, b1_p, w2_p, b2_p)
    h = _node_proj(node_feats, proj_W, proj_b.reshape(1, D_OUT))
    parts = None
    for step in range(2):
        hp = _gather(h, src3).reshape(E // 8, 128)
        m = _edge_messages(we_p, hp, q_p, g_p)
        parts = _scatter(m.reshape(E, D_OUT), dst3, zeros)
        if step == 0:
            h = _combine(parts[0], parts[1], conv_b.reshape(1, D_OUT))
    return _final(parts[0], parts[1], conv_b.reshape(1, D_OUT),
                  pred_W, pred_b.reshape(1, 2))


# speedup vs baseline: 6.7610x; 1.0380x over previous
import jax
import jax.numpy as jnp
from jax import lax
from jax.experimental import pallas as pl
from jax.experimental.pallas import tpu as pltpu
from jax.experimental.pallas import tpu_sc as plsc

N = 10000
E = 320000
D_IN = 128
D_EDGE = 16
D_OUT = 16
D_HID = 32
D_OO = D_OUT * D_OUT

NC = 2
NS = 16
NW = NC * NS
EW = E // NW
CG = 80
NCH = EW // CG
NPT = N // NS
SUP = 25
NSUP = NCH // SUP
SB = SUP * CG

_mesh = plsc.VectorSubcoreMesh(core_axis_name="c", subcore_axis_name="s")
_sc_params = pltpu.CompilerParams(use_tc_tiling_on_sc=False)


def _proj_body(nf_ref, w_ref, b_ref, o_ref):
    x = jnp.dot(nf_ref[...], w_ref[...], preferred_element_type=jnp.float32)
    o_ref[...] = jnp.maximum(x + b_ref[...], 0.0)


def _node_proj(nf, w, b2d):
    bm = 1000
    return pl.pallas_call(
        _proj_body,
        grid=(N // bm,),
        in_specs=[
            pl.BlockSpec((bm, D_IN), lambda i: (i, 0)),
            pl.BlockSpec((D_IN, D_OUT), lambda i: (0, 0)),
            pl.BlockSpec((1, D_OUT), lambda i: (0, 0)),
        ],
        out_specs=pl.BlockSpec((bm, D_OUT), lambda i: (i, 0)),
        out_shape=jax.ShapeDtypeStruct((N, D_OUT), jnp.float32),
    )(nf, w, b2d)


def _msg_body(ef_ref, hp_ref, w1_ref, b1_ref, w2p_ref, b2p_ref, q_ref, g_ref,
              o_ref):
    a = jnp.dot(ef_ref[...].astype(jnp.bfloat16), w1_ref[...],
                preferred_element_type=jnp.float32)
    a = jnp.maximum(a + b1_ref[...], 0.0).astype(jnp.bfloat16)
    we = jnp.dot(a, w2p_ref[...], preferred_element_type=jnp.float32)
    we = (we + b2p_ref[...]).astype(jnp.bfloat16)
    hpt = jnp.dot(hp_ref[...].astype(jnp.bfloat16), q_ref[...],
                  preferred_element_type=jnp.float32).astype(jnp.bfloat16)
    prod = we * hpt
    o_ref[...] = jnp.dot(prod, g_ref[...],
                         preferred_element_type=jnp.float32)


def _edge_messages(ef, hp, w1_p, b1_p, w2_p, b2_p, q_p, g_p):
    bm = 800
    ep8 = E // 8
    return pl.pallas_call(
        _msg_body,
        grid=(ep8 // bm,),
        in_specs=[
            pl.BlockSpec((bm, 128), lambda i: (i, 0)),
            pl.BlockSpec((bm, 128), lambda i: (i, 0)),
            pl.BlockSpec((128, 8 * D_HID), lambda i: (0, 0)),
            pl.BlockSpec((1, 8 * D_HID), lambda i: (0, 0)),
            pl.BlockSpec((8 * D_HID, 8 * D_OO), lambda i: (0, 0)),
            pl.BlockSpec((1, 8 * D_OO), lambda i: (0, 0)),
            pl.BlockSpec((128, 8 * D_OO), lambda i: (0, 0)),
            pl.BlockSpec((8 * D_OO, 128), lambda i: (0, 0)),
        ],
        out_specs=pl.BlockSpec((bm, 128), lambda i: (i, 0)),
        out_shape=jax.ShapeDtypeStruct((ep8, 128), jnp.float32),
    )(ef, hp, w1_p, b1_p, w2_p, b2_p, q_p, g_p)


def _combine_body(p0_ref, p1_ref, b_ref, o_ref):
    o_ref[...] = jnp.maximum(p0_ref[...] + p1_ref[...] + b_ref[...], 0.0)


def _combine(p0, p1, b2d):
    bm = 1000
    return pl.pallas_call(
        _combine_body,
        grid=(N // bm,),
        in_specs=[
            pl.BlockSpec((bm, D_OUT), lambda i: (i, 0)),
            pl.BlockSpec((bm, D_OUT), lambda i: (i, 0)),
            pl.BlockSpec((1, D_OUT), lambda i: (0, 0)),
        ],
        out_specs=pl.BlockSpec((bm, D_OUT), lambda i: (i, 0)),
        out_shape=jax.ShapeDtypeStruct((N, D_OUT), jnp.float32),
    )(p0, p1, b2d)


def _final_body(p0_ref, p1_ref, b_ref, pw_ref, pb_ref, o_ref):
    h = jnp.maximum(p0_ref[...] + p1_ref[...] + b_ref[...], 0.0)
    gm = jnp.sum(h, axis=0, keepdims=True) * (1.0 / N)
    o_ref[...] = jnp.dot(gm, pw_ref[...], preferred_element_type=jnp.float32) + pb_ref[...]


def _final(p0, p1, b2d, pw, pb2d):
    return pl.pallas_call(
        _final_body,
        in_specs=[
            pl.BlockSpec((N, D_OUT), lambda: (0, 0)),
            pl.BlockSpec((N, D_OUT), lambda: (0, 0)),
            pl.BlockSpec((1, D_OUT), lambda: (0, 0)),
            pl.BlockSpec((D_OUT, 2), lambda: (0, 0)),
            pl.BlockSpec((1, 2), lambda: (0, 0)),
        ],
        out_specs=pl.BlockSpec((1, 2), lambda: (0, 0)),
        out_shape=jax.ShapeDtypeStruct((1, 2), jnp.float32),
    )(p0, p1, b2d, pw, pb2d)


def _gather_body(h_hbm, src_hbm, out_hbm, idx_v, rows_v, sem):
    cid = lax.axis_index("c")
    sid = lax.axis_index("s")
    w = cid * NS + sid
    pltpu.sync_copy(src_hbm.at[w], idx_v)
    base = w * EW

    def sup(s, carry):
        descs = [
            pltpu.async_copy(
                h_hbm.at[idx_v.at[s * SUP + c]],
                rows_v.at[pl.ds(c * CG, CG)], sem)
            for c in range(SUP)
        ]
        for d in descs:
            d.wait()
        pltpu.sync_copy(rows_v, out_hbm.at[pl.ds(base + s * SB, SB)])
        return carry

    lax.fori_loop(0, NSUP, sup, 0)


def _gather(h, src3):
    return pl.kernel(
        _gather_body,
        out_type=jax.ShapeDtypeStruct((E, D_OUT), jnp.float32),
        mesh=_mesh,
        scratch_types=[
            pltpu.VMEM((NCH, CG), jnp.int32),
            pltpu.VMEM((SB, D_OUT), jnp.float32),
            pltpu.SemaphoreType.DMA,
        ],
        compiler_params=_sc_params,
    )(h, src3)


def _scatter_body(m_hbm, dst_hbm, z_hbm, out_hbm, agg_sh, idx_v, m_v, row_v, sem):
    cid = lax.axis_index("c")
    sid = lax.axis_index("s")
    w = cid * NS + sid
    pltpu.sync_copy(z_hbm.at[pl.ds(sid * NPT, NPT)], row_v)
    pltpu.sync_copy(row_v, agg_sh.at[pl.ds(sid * NPT, NPT)])
    pltpu.sync_copy(dst_hbm.at[w], idx_v)
    plsc.subcore_barrier()
    base = w * EW

    def sup(s, carry):
        pltpu.sync_copy(m_hbm.at[pl.ds(base + s * SB, SB)], m_v)
        descs = [
            pltpu.async_copy(
                m_v.at[pl.ds(c * CG, CG)],
                agg_sh.at[idx_v.at[s * SUP + c]], sem, add=True)
            for c in range(SUP)
        ]
        for d in descs:
            d.wait()
        return carry

    lax.fori_loop(0, NSUP, sup, 0)
    plsc.subcore_barrier()
    pltpu.sync_copy(agg_sh.at[pl.ds(sid * NPT, NPT)], row_v)
    pltpu.sync_copy(row_v, out_hbm.at[cid].at[pl.ds(sid * NPT, NPT)])


def _scatter(m, dst3, zeros):
    return pl.kernel(
        _scatter_body,
        out_type=jax.ShapeDtypeStruct((NC, N, D_OUT), jnp.float32),
        mesh=_mesh,
        scratch_types=[
            pltpu.VMEM_SHARED((N, D_OUT), jnp.float32),
            pltpu.VMEM((NCH, CG), jnp.int32),
            pltpu.VMEM((SB, D_OUT), jnp.float32),
            pltpu.VMEM((NPT, D_OUT), jnp.float32),
            pltpu.SemaphoreType.DMA,
        ],
        compiler_params=_sc_params,
    )(m, dst3, zeros)


def kernel(node_feats, edge_feats, edge_index, proj_W, proj_b, e1_W, e1_b,
           e2_W, e2_b, conv_b, pred_W, pred_b):
    src3 = edge_index[0].astype(jnp.int32).reshape(NW, NCH, CG)
    dst3 = edge_index[1].astype(jnp.int32).reshape(NW, NCH, CG)

    j = jnp.arange(D_OO)
    perm = (j % D_OUT) * D_OUT + j // D_OUT
    w2perm = e2_W[:, perm]
    b2perm = e2_b[perm]
    eye8 = jnp.eye(8, dtype=jnp.float32)
    w1_p = jnp.kron(eye8, e1_W).astype(jnp.bfloat16)
    b1_p = jnp.tile(e1_b, 8).reshape(1, 8 * D_HID)
    w2_p = jnp.kron(eye8, w2perm).astype(jnp.bfloat16)
    b2_p = jnp.tile(b2perm, 8).reshape(1, 8 * D_OO)
    q16 = jnp.concatenate([jnp.eye(D_OUT, dtype=jnp.float32)] * D_OUT, axis=1)
    q_p = jnp.kron(eye8, q16).astype(jnp.bfloat16)
    g16 = jnp.repeat(jnp.eye(D_OUT, dtype=jnp.float32), D_OUT, axis=0)
    g_p = jnp.kron(eye8, g16).astype(jnp.bfloat16)
    zeros = jnp.zeros((N, D_OUT), jnp.float32)
    ef_p = edge_feats.reshape(E // 8, 128)

    h = _node_proj(node_feats, proj_W, proj_b.reshape(1, D_OUT))
    parts = None
    for step in range(2):
        hp = _gather(h, src3).reshape(E // 8, 128)
        m = _edge_messages(ef_p, hp, w1_p, b1_p, w2_p, b2_p, q_p, g_p)
        parts = _scatter(m.reshape(E, D_OUT), dst3, zeros)
        if step == 0:
            h = _combine(parts[0], parts[1], conv_b.reshape(1, D_OUT))
    return _final(parts[0], parts[1], conv_b.reshape(1, D_OUT),
                  pred_W, pred_b.reshape(1, 2))


# speedup vs baseline: 6.8996x; 1.0205x over previous
import jax
import jax.numpy as jnp
from jax import lax
from jax.experimental import pallas as pl
from jax.experimental.pallas import tpu as pltpu
from jax.experimental.pallas import tpu_sc as plsc

N = 10000
E = 320000
D_IN = 128
D_EDGE = 16
D_OUT = 16
D_HID = 32
D_OO = D_OUT * D_OUT

NC = 2
NS = 16
NW = NC * NS
EW = E // NW
CG = 80
NCH = EW // CG
NPT = N // NS
SUP = 25
NSUP = NCH // SUP
SB = SUP * CG

_mesh = plsc.VectorSubcoreMesh(core_axis_name="c", subcore_axis_name="s")
_sc_params = pltpu.CompilerParams(use_tc_tiling_on_sc=False)


def _proj_body(nf_ref, w_ref, b_ref, o_ref):
    x = jnp.dot(nf_ref[...], w_ref[...], preferred_element_type=jnp.float32)
    o_ref[...] = jnp.maximum(x + b_ref[...], 0.0)


def _node_proj(nf, w, b2d):
    bm = 1000
    return pl.pallas_call(
        _proj_body,
        grid=(N // bm,),
        in_specs=[
            pl.BlockSpec((bm, D_IN), lambda i: (i, 0)),
            pl.BlockSpec((D_IN, D_OUT), lambda i: (0, 0)),
            pl.BlockSpec((1, D_OUT), lambda i: (0, 0)),
        ],
        out_specs=pl.BlockSpec((bm, D_OUT), lambda i: (i, 0)),
        out_shape=jax.ShapeDtypeStruct((N, D_OUT), jnp.float32),
    )(nf, w, b2d)


def _msg_body(ef_ref, hp_ref, w1_ref, b1_ref, w2p_ref, b2p_ref, q_ref, g_ref,
              o_ref):
    a = jnp.dot(ef_ref[...].astype(jnp.bfloat16), w1_ref[...],
                preferred_element_type=jnp.float32)
    a = jnp.maximum(a + b1_ref[...], 0.0).astype(jnp.bfloat16)
    we = jnp.dot(a, w2p_ref[...], preferred_element_type=jnp.float32)
    we = we + b2p_ref[...]
    hpt = jnp.dot(hp_ref[...].astype(jnp.bfloat16), q_ref[...],
                  preferred_element_type=jnp.float32)
    prod = (we * hpt).astype(jnp.bfloat16)
    o_ref[...] = jnp.dot(prod, g_ref[...],
                         preferred_element_type=jnp.float32)


def _edge_messages(ef, hp, w1_p, b1_p, w2_p, b2_p, q_p, g_p):
    bm = 800
    ep8 = E // 8
    return pl.pallas_call(
        _msg_body,
        grid=(ep8 // bm,),
        in_specs=[
            pl.BlockSpec((bm, 128), lambda i: (i, 0)),
            pl.BlockSpec((bm, 128), lambda i: (i, 0)),
            pl.BlockSpec((128, 8 * D_HID), lambda i: (0, 0)),
            pl.BlockSpec((1, 8 * D_HID), lambda i: (0, 0)),
            pl.BlockSpec((8 * D_HID, 8 * D_OO), lambda i: (0, 0)),
            pl.BlockSpec((1, 8 * D_OO), lambda i: (0, 0)),
            pl.BlockSpec((128, 8 * D_OO), lambda i: (0, 0)),
            pl.BlockSpec((8 * D_OO, 128), lambda i: (0, 0)),
        ],
        out_specs=pl.BlockSpec((bm, 128), lambda i: (i, 0)),
        out_shape=jax.ShapeDtypeStruct((ep8, 128), jnp.float32),
    )(ef, hp, w1_p, b1_p, w2_p, b2_p, q_p, g_p)


def _combine_body(p_ref, b_ref, o_ref):
    o_ref[...] = jnp.maximum(p_ref[0] + p_ref[1] + b_ref[...], 0.0)


def _combine(parts, b2d):
    bm = 1000
    return pl.pallas_call(
        _combine_body,
        grid=(N // bm,),
        in_specs=[
            pl.BlockSpec((NC, bm, D_OUT), lambda i: (0, i, 0)),
            pl.BlockSpec((1, D_OUT), lambda i: (0, 0)),
        ],
        out_specs=pl.BlockSpec((bm, D_OUT), lambda i: (i, 0)),
        out_shape=jax.ShapeDtypeStruct((N, D_OUT), jnp.float32),
    )(parts, b2d)


def _final_body(p_ref, b_ref, pw_ref, pb_ref, o_ref):
    h = jnp.maximum(p_ref[0] + p_ref[1] + b_ref[...], 0.0)
    gm = jnp.sum(h, axis=0, keepdims=True) * (1.0 / N)
    o_ref[...] = jnp.dot(gm, pw_ref[...], preferred_element_type=jnp.float32) + pb_ref[...]


def _final(parts, b2d, pw, pb2d):
    return pl.pallas_call(
        _final_body,
        in_specs=[
            pl.BlockSpec((NC, N, D_OUT), lambda: (0, 0, 0)),
            pl.BlockSpec((1, D_OUT), lambda: (0, 0)),
            pl.BlockSpec((D_OUT, 2), lambda: (0, 0)),
            pl.BlockSpec((1, 2), lambda: (0, 0)),
        ],
        out_specs=pl.BlockSpec((1, 2), lambda: (0, 0)),
        out_shape=jax.ShapeDtypeStruct((1, 2), jnp.float32),
    )(parts, b2d, pw, pb2d)


def _gather_body(h_hbm, src_hbm, out_hbm, idx_v, rows_v, sem, sem2):
    cid = lax.axis_index("c")
    sid = lax.axis_index("s")
    w = cid * NS + sid
    pltpu.sync_copy(src_hbm.at[w], idx_v)
    base = w * EW

    def fire(s, b):
        return [
            pltpu.async_copy(
                h_hbm.at[idx_v.at[s * SUP + c]],
                rows_v.at[b, pl.ds(c * CG, CG)], sem)
            for c in range(SUP)
        ]

    outs = []
    descs = fire(0, 0)
    for s in range(NSUP):
        for d in descs:
            d.wait()
        if outs:
            outs.pop(0).wait()
        if s + 1 < NSUP:
            descs = fire(s + 1, (s + 1) % 2)
        outs.append(pltpu.async_copy(
            rows_v.at[s % 2], out_hbm.at[pl.ds(base + s * SB, SB)], sem2))
    outs.pop(0).wait()


def _gather(h, src3):
    return pl.kernel(
        _gather_body,
        out_type=jax.ShapeDtypeStruct((E, D_OUT), jnp.float32),
        mesh=_mesh,
        scratch_types=[
            pltpu.VMEM((NCH, CG), jnp.int32),
            pltpu.VMEM((2, SB, D_OUT), jnp.float32),
            pltpu.SemaphoreType.DMA,
            pltpu.SemaphoreType.DMA,
        ],
        compiler_params=_sc_params,
    )(h, src3)


def _scatter_body(m_hbm, dst_hbm, z_hbm, out_hbm, agg_sh, idx_v, m_v, row_v,
                  seml, sems):
    cid = lax.axis_index("c")
    sid = lax.axis_index("s")
    w = cid * NS + sid
    pltpu.sync_copy(z_hbm.at[pl.ds(sid * NPT, NPT)], row_v)
    pltpu.sync_copy(row_v, agg_sh.at[pl.ds(sid * NPT, NPT)])
    pltpu.sync_copy(dst_hbm.at[w], idx_v)
    plsc.subcore_barrier()
    base = w * EW

    def load(s):
        return pltpu.async_copy(
            m_hbm.at[pl.ds(base + s * SB, SB)], m_v.at[s % 2], seml)

    ld = load(0)
    prev = []
    for s in range(NSUP):
        ld.wait()
        for d in prev:
            d.wait()
        if s + 1 < NSUP:
            ld = load(s + 1)
        prev = [
            pltpu.async_copy(
                m_v.at[s % 2, pl.ds(c * CG, CG)],
                agg_sh.at[idx_v.at[s * SUP + c]], sems, add=True)
            for c in range(SUP)
        ]
    for d in prev:
        d.wait()
    plsc.subcore_barrier()
    pltpu.sync_copy(agg_sh.at[pl.ds(sid * NPT, NPT)], row_v)
    pltpu.sync_copy(row_v, out_hbm.at[cid].at[pl.ds(sid * NPT, NPT)])


def _scatter(m, dst3, zeros):
    return pl.kernel(
        _scatter_body,
        out_type=jax.ShapeDtypeStruct((NC, N, D_OUT), jnp.float32),
        mesh=_mesh,
        scratch_types=[
            pltpu.VMEM_SHARED((N, D_OUT), jnp.float32),
            pltpu.VMEM((NCH, CG), jnp.int32),
            pltpu.VMEM((2, SB, D_OUT), jnp.float32),
            pltpu.VMEM((NPT, D_OUT), jnp.float32),
            pltpu.SemaphoreType.DMA,
            pltpu.SemaphoreType.DMA,
        ],
        compiler_params=_sc_params,
    )(m, dst3, zeros)


def kernel(node_feats, edge_feats, edge_index, proj_W, proj_b, e1_W, e1_b,
           e2_W, e2_b, conv_b, pred_W, pred_b):
    src3 = edge_index[0].astype(jnp.int32).reshape(NW, NCH, CG)
    dst3 = edge_index[1].astype(jnp.int32).reshape(NW, NCH, CG)

    j = jnp.arange(D_OO)
    perm = (j % D_OUT) * D_OUT + j // D_OUT
    w2perm = e2_W[:, perm]
    b2perm = e2_b[perm]
    eye8 = jnp.eye(8, dtype=jnp.float32)
    w1_p = jnp.kron(eye8, e1_W).astype(jnp.bfloat16)
    b1_p = jnp.tile(e1_b, 8).reshape(1, 8 * D_HID)
    w2_p = jnp.kron(eye8, w2perm).astype(jnp.bfloat16)
    b2_p = jnp.tile(b2perm, 8).reshape(1, 8 * D_OO)
    q16 = jnp.concatenate([jnp.eye(D_OUT, dtype=jnp.float32)] * D_OUT, axis=1)
    q_p = jnp.kron(eye8, q16).astype(jnp.bfloat16)
    g16 = jnp.repeat(jnp.eye(D_OUT, dtype=jnp.float32), D_OUT, axis=0)
    g_p = jnp.kron(eye8, g16).astype(jnp.bfloat16)
    zeros = jnp.zeros((N, D_OUT), jnp.float32)
    ef_p = edge_feats.reshape(E // 8, 128)

    h = _node_proj(node_feats, proj_W, proj_b.reshape(1, D_OUT))
    parts = None
    for step in range(2):
        hp = _gather(h, src3).reshape(E // 8, 128)
        m = _edge_messages(ef_p, hp, w1_p, b1_p, w2_p, b2_p, q_p, g_p)
        parts = _scatter(m.reshape(E, D_OUT), dst3, zeros)
        if step == 0:
            h = _combine(parts, conv_b.reshape(1, D_OUT))
    return _final(parts, conv_b.reshape(1, D_OUT),
                  pred_W, pred_b.reshape(1, 2))


# speedup vs baseline: 7.0514x; 1.0220x over previous
import jax
import jax.numpy as jnp
from jax import lax
from jax.experimental import pallas as pl
from jax.experimental.pallas import tpu as pltpu
from jax.experimental.pallas import tpu_sc as plsc

N = 10000
E = 320000
D_IN = 128
D_EDGE = 16
D_OUT = 16
D_HID = 32
D_OO = D_OUT * D_OUT

NC = 2
NS = 16
NW = NC * NS
EW = E // NW
CG = 80
NCH = EW // CG
NPT = N // NS
SUP = 25
NSUP = NCH // SUP
SB = SUP * CG

_mesh = plsc.VectorSubcoreMesh(core_axis_name="c", subcore_axis_name="s")
_sc_params = pltpu.CompilerParams(use_tc_tiling_on_sc=False)


def _proj_body(nf_ref, w_ref, b_ref, o_ref):
    x = jnp.dot(nf_ref[...], w_ref[...], preferred_element_type=jnp.float32)
    o_ref[...] = jnp.maximum(x + b_ref[...], 0.0)


def _node_proj(nf, w, b2d):
    bm = 1000
    return pl.pallas_call(
        _proj_body,
        grid=(N // bm,),
        in_specs=[
            pl.BlockSpec((bm, D_IN), lambda i: (i, 0)),
            pl.BlockSpec((D_IN, D_OUT), lambda i: (0, 0)),
            pl.BlockSpec((1, D_OUT), lambda i: (0, 0)),
        ],
        out_specs=pl.BlockSpec((bm, D_OUT), lambda i: (i, 0)),
        out_shape=jax.ShapeDtypeStruct((N, D_OUT), jnp.float32),
    )(nf, w, b2d)


def _msg_body(ef_ref, hp_ref, w1_ref, b1_ref, w2p_ref, b2p_ref, q_ref, g_ref,
              o_ref):
    a = jnp.dot(ef_ref[...].astype(jnp.bfloat16), w1_ref[...],
                preferred_element_type=jnp.float32)
    a = jnp.maximum(a + b1_ref[...], 0.0).astype(jnp.bfloat16)
    we = jnp.dot(a, w2p_ref[...], preferred_element_type=jnp.float32)
    we = we + b2p_ref[...]
    hpt = jnp.dot(hp_ref[...].astype(jnp.bfloat16), q_ref[...],
                  preferred_element_type=jnp.float32)
    prod = (we * hpt).astype(jnp.bfloat16)
    o_ref[...] = jnp.dot(prod, g_ref[...],
                         preferred_element_type=jnp.float32)


def _edge_messages(ef, hp, w1_p, b1_p, w2_p, b2_p, q_p, g_p):
    bm = 1600
    ep8 = E // 8
    return pl.pallas_call(
        _msg_body,
        grid=(ep8 // bm,),
        in_specs=[
            pl.BlockSpec((bm, 128), lambda i: (i, 0)),
            pl.BlockSpec((bm, 128), lambda i: (i, 0)),
            pl.BlockSpec((128, 8 * D_HID), lambda i: (0, 0)),
            pl.BlockSpec((1, 8 * D_HID), lambda i: (0, 0)),
            pl.BlockSpec((8 * D_HID, 8 * D_OO), lambda i: (0, 0)),
            pl.BlockSpec((1, 8 * D_OO), lambda i: (0, 0)),
            pl.BlockSpec((128, 8 * D_OO), lambda i: (0, 0)),
            pl.BlockSpec((8 * D_OO, 128), lambda i: (0, 0)),
        ],
        out_specs=pl.BlockSpec((bm, 128), lambda i: (i, 0)),
        out_shape=jax.ShapeDtypeStruct((ep8, 128), jnp.float32),
    )(ef, hp, w1_p, b1_p, w2_p, b2_p, q_p, g_p)


def _combine_body(p_ref, b_ref, o_ref):
    o_ref[...] = jnp.maximum(p_ref[0] + p_ref[1] + b_ref[...], 0.0)


def _combine(parts, b2d):
    bm = 1000
    return pl.pallas_call(
        _combine_body,
        grid=(N // bm,),
        in_specs=[
            pl.BlockSpec((NC, bm, D_OUT), lambda i: (0, i, 0)),
            pl.BlockSpec((1, D_OUT), lambda i: (0, 0)),
        ],
        out_specs=pl.BlockSpec((bm, D_OUT), lambda i: (i, 0)),
        out_shape=jax.ShapeDtypeStruct((N, D_OUT), jnp.float32),
    )(parts, b2d)


def _final_body(p_ref, b_ref, pw_ref, pb_ref, o_ref):
    h = jnp.maximum(p_ref[0] + p_ref[1] + b_ref[...], 0.0)
    gm = jnp.sum(h, axis=0, keepdims=True) * (1.0 / N)
    o_ref[...] = jnp.dot(gm, pw_ref[...], preferred_element_type=jnp.float32) + pb_ref[...]


def _final(parts, b2d, pw, pb2d):
    return pl.pallas_call(
        _final_body,
        in_specs=[
            pl.BlockSpec((NC, N, D_OUT), lambda: (0, 0, 0)),
            pl.BlockSpec((1, D_OUT), lambda: (0, 0)),
            pl.BlockSpec((D_OUT, 2), lambda: (0, 0)),
            pl.BlockSpec((1, 2), lambda: (0, 0)),
        ],
        out_specs=pl.BlockSpec((1, 2), lambda: (0, 0)),
        out_shape=jax.ShapeDtypeStruct((1, 2), jnp.float32),
    )(parts, b2d, pw, pb2d)


def _gather_body(h_hbm, src_hbm, out_hbm, idx_v, rows_v, sem, sem2):
    cid = lax.axis_index("c")
    sid = lax.axis_index("s")
    w = cid * NS + sid
    pltpu.sync_copy(src_hbm.at[w], idx_v)
    base = w * EW

    def fire(s, b):
        return [
            pltpu.async_copy(
                h_hbm.at[idx_v.at[s * SUP + c]],
                rows_v.at[b, pl.ds(c * CG, CG)], sem)
            for c in range(SUP)
        ]

    outs = []
    descs = fire(0, 0)
    for s in range(NSUP):
        for d in descs:
            d.wait()
        if outs:
            outs.pop(0).wait()
        if s + 1 < NSUP:
            descs = fire(s + 1, (s + 1) % 2)
        outs.append(pltpu.async_copy(
            rows_v.at[s % 2], out_hbm.at[pl.ds(base + s * SB, SB)], sem2))
    outs.pop(0).wait()


def _gather(h, src3):
    return pl.kernel(
        _gather_body,
        out_type=jax.ShapeDtypeStruct((E, D_OUT), jnp.float32),
        mesh=_mesh,
        scratch_types=[
            pltpu.VMEM((NCH, CG), jnp.int32),
            pltpu.VMEM((2, SB, D_OUT), jnp.float32),
            pltpu.SemaphoreType.DMA,
            pltpu.SemaphoreType.DMA,
        ],
        compiler_params=_sc_params,
    )(h, src3)


def _scatter_body(m_hbm, dst_hbm, z_hbm, out_hbm, agg_sh, idx_v, m_v, row_v,
                  seml, sems):
    cid = lax.axis_index("c")
    sid = lax.axis_index("s")
    w = cid * NS + sid
    pltpu.sync_copy(z_hbm.at[pl.ds(sid * NPT, NPT)], row_v)
    pltpu.sync_copy(row_v, agg_sh.at[pl.ds(sid * NPT, NPT)])
    pltpu.sync_copy(dst_hbm.at[w], idx_v)
    plsc.subcore_barrier()
    base = w * EW

    def load(s):
        return pltpu.async_copy(
            m_hbm.at[pl.ds(base + s * SB, SB)], m_v.at[s % 2], seml)

    ld = load(0)
    prev = []
    for s in range(NSUP):
        ld.wait()
        for d in prev:
            d.wait()
        if s + 1 < NSUP:
            ld = load(s + 1)
        prev = [
            pltpu.async_copy(
                m_v.at[s % 2, pl.ds(c * CG, CG)],
                agg_sh.at[idx_v.at[s * SUP + c]], sems, add=True)
            for c in range(SUP)
        ]
    for d in prev:
        d.wait()
    plsc.subcore_barrier()
    pltpu.sync_copy(agg_sh.at[pl.ds(sid * NPT, NPT)], row_v)
    pltpu.sync_copy(row_v, out_hbm.at[cid].at[pl.ds(sid * NPT, NPT)])


def _scatter(m, dst3, zeros):
    return pl.kernel(
        _scatter_body,
        out_type=jax.ShapeDtypeStruct((NC, N, D_OUT), jnp.float32),
        mesh=_mesh,
        scratch_types=[
            pltpu.VMEM_SHARED((N, D_OUT), jnp.float32),
            pltpu.VMEM((NCH, CG), jnp.int32),
            pltpu.VMEM((2, SB, D_OUT), jnp.float32),
            pltpu.VMEM((NPT, D_OUT), jnp.float32),
            pltpu.SemaphoreType.DMA,
            pltpu.SemaphoreType.DMA,
        ],
        compiler_params=_sc_params,
    )(m, dst3, zeros)


def kernel(node_feats, edge_feats, edge_index, proj_W, proj_b, e1_W, e1_b,
           e2_W, e2_b, conv_b, pred_W, pred_b):
    src3 = edge_index[0].astype(jnp.int32).reshape(NW, NCH, CG)
    dst3 = edge_index[1].astype(jnp.int32).reshape(NW, NCH, CG)

    j = jnp.arange(D_OO)
    perm = (j % D_OUT) * D_OUT + j // D_OUT
    w2perm = e2_W[:, perm]
    b2perm = e2_b[perm]
    eye8 = jnp.eye(8, dtype=jnp.float32)
    w1_p = jnp.kron(eye8, e1_W).astype(jnp.bfloat16)
    b1_p = jnp.tile(e1_b, 8).reshape(1, 8 * D_HID)
    w2_p = jnp.kron(eye8, w2perm).astype(jnp.bfloat16)
    b2_p = jnp.tile(b2perm, 8).reshape(1, 8 * D_OO)
    q16 = jnp.concatenate([jnp.eye(D_OUT, dtype=jnp.float32)] * D_OUT, axis=1)
    q_p = jnp.kron(eye8, q16).astype(jnp.bfloat16)
    g16 = jnp.repeat(jnp.eye(D_OUT, dtype=jnp.float32), D_OUT, axis=0)
    g_p = jnp.kron(eye8, g16).astype(jnp.bfloat16)
    zeros = jnp.zeros((N, D_OUT), jnp.float32)
    ef_p = edge_feats.reshape(E // 8, 128)

    h = _node_proj(node_feats, proj_W, proj_b.reshape(1, D_OUT))
    parts = None
    for step in range(2):
        hp = _gather(h, src3).reshape(E // 8, 128)
        m = _edge_messages(ef_p, hp, w1_p, b1_p, w2_p, b2_p, q_p, g_p)
        parts = _scatter(m.reshape(E, D_OUT), dst3, zeros)
        if step == 0:
            h = _combine(parts, conv_b.reshape(1, D_OUT))
    return _final(parts, conv_b.reshape(1, D_OUT),
                  pred_W, pred_b.reshape(1, 2))


# speedup vs baseline: 7.3690x; 1.0450x over previous
import jax
import jax.numpy as jnp
from jax import lax
from jax.experimental import pallas as pl
from jax.experimental.pallas import tpu as pltpu
from jax.experimental.pallas import tpu_sc as plsc

N = 10000
E = 320000
D_IN = 128
D_EDGE = 16
D_OUT = 16
D_HID = 32
D_OO = D_OUT * D_OUT

NC = 2
NS = 16
NW = NC * NS
EW = E // NW
CG = 80
NCH = EW // CG
NPT = N // NS
SUP = 25
NSUP = NCH // SUP
SB = SUP * CG

_mesh = plsc.VectorSubcoreMesh(core_axis_name="c", subcore_axis_name="s")
_sc_params = pltpu.CompilerParams(use_tc_tiling_on_sc=False)


def _proj_body(nf_ref, w_ref, b_ref, o_ref):
    x = jnp.dot(nf_ref[...], w_ref[...], preferred_element_type=jnp.float32)
    o_ref[...] = jnp.maximum(x + b_ref[...], 0.0)


def _node_proj(nf, w, b2d):
    bm = 1000
    return pl.pallas_call(
        _proj_body,
        grid=(N // bm,),
        in_specs=[
            pl.BlockSpec((bm, D_IN), lambda i: (i, 0)),
            pl.BlockSpec((D_IN, D_OUT), lambda i: (0, 0)),
            pl.BlockSpec((1, D_OUT), lambda i: (0, 0)),
        ],
        out_specs=pl.BlockSpec((bm, D_OUT), lambda i: (i, 0)),
        out_shape=jax.ShapeDtypeStruct((N, D_OUT), jnp.float32),
    )(nf, w, b2d)


def _msg_body(ef_ref, hp_ref, w1_ref, b1_ref, w2p_ref, b2p_ref, q_ref, g_ref,
              o_ref):
    a = jnp.dot(ef_ref[...].astype(jnp.bfloat16), w1_ref[...],
                preferred_element_type=jnp.float32)
    a = jnp.maximum(a + b1_ref[...], 0.0).astype(jnp.bfloat16)
    we = jnp.dot(a, w2p_ref[...], preferred_element_type=jnp.float32)
    we = we + b2p_ref[...]
    hpt = jnp.dot(hp_ref[...].astype(jnp.bfloat16), q_ref[...],
                  preferred_element_type=jnp.float32)
    prod = (we * hpt).astype(jnp.bfloat16)
    o_ref[...] = jnp.dot(prod, g_ref[...],
                         preferred_element_type=jnp.float32)


def _edge_messages(ef, hp, w1_p, b1_p, w2_p, b2_p, q_p, g_p):
    bm = 1600
    ep8 = E // 8
    return pl.pallas_call(
        _msg_body,
        grid=(ep8 // bm,),
        in_specs=[
            pl.BlockSpec((bm, 128), lambda i: (i, 0)),
            pl.BlockSpec((bm, 128), lambda i: (i, 0)),
            pl.BlockSpec((128, 8 * D_HID), lambda i: (0, 0)),
            pl.BlockSpec((1, 8 * D_HID), lambda i: (0, 0)),
            pl.BlockSpec((8 * D_HID, 8 * D_OO), lambda i: (0, 0)),
            pl.BlockSpec((1, 8 * D_OO), lambda i: (0, 0)),
            pl.BlockSpec((128, 8 * D_OO), lambda i: (0, 0)),
            pl.BlockSpec((8 * D_OO, 128), lambda i: (0, 0)),
        ],
        out_specs=pl.BlockSpec((bm, 128), lambda i: (i, 0)),
        out_shape=jax.ShapeDtypeStruct((ep8, 128), jnp.float32),
    )(ef, hp, w1_p, b1_p, w2_p, b2_p, q_p, g_p)


def _combine_body(p_ref, b_ref, o_ref):
    o_ref[...] = jnp.maximum(p_ref[0] + p_ref[1] + b_ref[...], 0.0)


def _combine(parts, b2d):
    bm = 1000
    return pl.pallas_call(
        _combine_body,
        grid=(N // bm,),
        in_specs=[
            pl.BlockSpec((NC, bm, D_OUT), lambda i: (0, i, 0)),
            pl.BlockSpec((1, D_OUT), lambda i: (0, 0)),
        ],
        out_specs=pl.BlockSpec((bm, D_OUT), lambda i: (i, 0)),
        out_shape=jax.ShapeDtypeStruct((N, D_OUT), jnp.float32),
    )(parts, b2d)


def _final_body(p_ref, b_ref, pw_ref, pb_ref, o_ref):
    h = jnp.maximum(p_ref[0] + p_ref[1] + b_ref[...], 0.0)
    gm = jnp.sum(h, axis=0, keepdims=True) * (1.0 / N)
    o_ref[...] = jnp.dot(gm, pw_ref[...], preferred_element_type=jnp.float32) + pb_ref[...]


def _final(parts, b2d, pw, pb2d):
    return pl.pallas_call(
        _final_body,
        in_specs=[
            pl.BlockSpec((NC, N, D_OUT), lambda: (0, 0, 0)),
            pl.BlockSpec((1, D_OUT), lambda: (0, 0)),
            pl.BlockSpec((D_OUT, 2), lambda: (0, 0)),
            pl.BlockSpec((1, 2), lambda: (0, 0)),
        ],
        out_specs=pl.BlockSpec((1, 2), lambda: (0, 0)),
        out_shape=jax.ShapeDtypeStruct((1, 2), jnp.float32),
    )(parts, b2d, pw, pb2d)


def _gather_body(h_hbm, src_hbm, out_hbm, idx_v, rows_v, sem, sem2):
    cid = lax.axis_index("c")
    sid = lax.axis_index("s")
    w = cid * NS + sid
    pltpu.sync_copy(src_hbm.at[w], idx_v)
    base = w * EW

    def fire(s, b):
        return [
            pltpu.async_copy(
                h_hbm.at[idx_v.at[s * SUP + c]],
                rows_v.at[b, pl.ds(c * CG, CG)], sem)
            for c in range(SUP)
        ]

    outs = []
    descs = fire(0, 0)
    for s in range(NSUP):
        for d in descs:
            d.wait()
        if outs:
            outs.pop(0).wait()
        if s + 1 < NSUP:
            descs = fire(s + 1, (s + 1) % 2)
        outs.append(pltpu.async_copy(
            rows_v.at[s % 2], out_hbm.at[pl.ds(base + s * SB, SB)], sem2))
    outs.pop(0).wait()


def _gather(h, src3):
    return pl.kernel(
        _gather_body,
        out_type=jax.ShapeDtypeStruct((E, D_OUT), jnp.float32),
        mesh=_mesh,
        scratch_types=[
            pltpu.VMEM((NCH, CG), jnp.int32),
            pltpu.VMEM((2, SB, D_OUT), jnp.float32),
            pltpu.SemaphoreType.DMA,
            pltpu.SemaphoreType.DMA,
        ],
        compiler_params=_sc_params,
    )(h, src3)


def _cgather_body(p_hbm, b_hbm, src_hbm, out_hbm, h_sh, idx_v, rows_v, pv0,
                  pv1, hbuf, bv, sem, sem2):
    cid = lax.axis_index("c")
    sid = lax.axis_index("s")
    w = cid * NS + sid
    nbase = sid * NPT
    pltpu.sync_copy(p_hbm.at[0].at[pl.ds(nbase, NPT)], pv0)
    pltpu.sync_copy(p_hbm.at[1].at[pl.ds(nbase, NPT)], pv1)
    pltpu.sync_copy(b_hbm, bv)
    pltpu.sync_copy(src_hbm.at[w], idx_v)

    def crow(r, carry):
        hbuf[r, :] = jnp.maximum(pv0[r, :] + pv1[r, :] + bv[0, :], 0.0)
        return carry

    lax.fori_loop(0, NPT, crow, 0)
    pltpu.sync_copy(hbuf, h_sh.at[pl.ds(nbase, NPT)])
    plsc.subcore_barrier()
    base = w * EW

    def fire(s, b):
        return [
            pltpu.async_copy(
                h_sh.at[idx_v.at[s * SUP + c]],
                rows_v.at[b, pl.ds(c * CG, CG)], sem)
            for c in range(SUP)
        ]

    outs = []
    descs = fire(0, 0)
    for s in range(NSUP):
        for d in descs:
            d.wait()
        if outs:
            outs.pop(0).wait()
        if s + 1 < NSUP:
            descs = fire(s + 1, (s + 1) % 2)
        outs.append(pltpu.async_copy(
            rows_v.at[s % 2], out_hbm.at[pl.ds(base + s * SB, SB)], sem2))
    outs.pop(0).wait()


def _combine_gather(parts, b2d, src3):
    return pl.kernel(
        _cgather_body,
        out_type=jax.ShapeDtypeStruct((E, D_OUT), jnp.float32),
        mesh=_mesh,
        scratch_types=[
            pltpu.VMEM_SHARED((N, D_OUT), jnp.float32),
            pltpu.VMEM((NCH, CG), jnp.int32),
            pltpu.VMEM((2, SB, D_OUT), jnp.float32),
            pltpu.VMEM((NPT, D_OUT), jnp.float32),
            pltpu.VMEM((NPT, D_OUT), jnp.float32),
            pltpu.VMEM((NPT, D_OUT), jnp.float32),
            pltpu.VMEM((1, D_OUT), jnp.float32),
            pltpu.SemaphoreType.DMA,
            pltpu.SemaphoreType.DMA,
        ],
        compiler_params=_sc_params,
    )(parts, b2d, src3)


def _scatter_body(m_hbm, dst_hbm, z_hbm, out_hbm, agg_sh, idx_v, m_v, row_v,
                  seml, sems):
    cid = lax.axis_index("c")
    sid = lax.axis_index("s")
    w = cid * NS + sid
    pltpu.sync_copy(z_hbm.at[pl.ds(sid * NPT, NPT)], row_v)
    pltpu.sync_copy(row_v, agg_sh.at[pl.ds(sid * NPT, NPT)])
    pltpu.sync_copy(dst_hbm.at[w], idx_v)
    plsc.subcore_barrier()
    base = w * EW

    def load(s):
        return pltpu.async_copy(
            m_hbm.at[pl.ds(base + s * SB, SB)], m_v.at[s % 2], seml)

    ld = load(0)
    prev = []
    for s in range(NSUP):
        ld.wait()
        for d in prev:
            d.wait()
        if s + 1 < NSUP:
            ld = load(s + 1)
        prev = [
            pltpu.async_copy(
                m_v.at[s % 2, pl.ds(c * CG, CG)],
                agg_sh.at[idx_v.at[s * SUP + c]], sems, add=True)
            for c in range(SUP)
        ]
    for d in prev:
        d.wait()
    plsc.subcore_barrier()
    pltpu.sync_copy(agg_sh.at[pl.ds(sid * NPT, NPT)], row_v)
    pltpu.sync_copy(row_v, out_hbm.at[cid].at[pl.ds(sid * NPT, NPT)])


def _scatter(m, dst3, zeros):
    return pl.kernel(
        _scatter_body,
        out_type=jax.ShapeDtypeStruct((NC, N, D_OUT), jnp.float32),
        mesh=_mesh,
        scratch_types=[
            pltpu.VMEM_SHARED((N, D_OUT), jnp.float32),
            pltpu.VMEM((NCH, CG), jnp.int32),
            pltpu.VMEM((2, SB, D_OUT), jnp.float32),
            pltpu.VMEM((NPT, D_OUT), jnp.float32),
            pltpu.SemaphoreType.DMA,
            pltpu.SemaphoreType.DMA,
        ],
        compiler_params=_sc_params,
    )(m, dst3, zeros)


def kernel(node_feats, edge_feats, edge_index, proj_W, proj_b, e1_W, e1_b,
           e2_W, e2_b, conv_b, pred_W, pred_b):
    src3 = edge_index[0].astype(jnp.int32).reshape(NW, NCH, CG)
    dst3 = edge_index[1].astype(jnp.int32).reshape(NW, NCH, CG)

    j = jnp.arange(D_OO)
    perm = (j % D_OUT) * D_OUT + j // D_OUT
    w2perm = e2_W[:, perm]
    b2perm = e2_b[perm]
    eye8 = jnp.eye(8, dtype=jnp.float32)
    w1_p = jnp.kron(eye8, e1_W).astype(jnp.bfloat16)
    b1_p = jnp.tile(e1_b, 8).reshape(1, 8 * D_HID)
    w2_p = jnp.kron(eye8, w2perm).astype(jnp.bfloat16)
    b2_p = jnp.tile(b2perm, 8).reshape(1, 8 * D_OO)
    q16 = jnp.concatenate([jnp.eye(D_OUT, dtype=jnp.float32)] * D_OUT, axis=1)
    q_p = jnp.kron(eye8, q16).astype(jnp.bfloat16)
    g16 = jnp.repeat(jnp.eye(D_OUT, dtype=jnp.float32), D_OUT, axis=0)
    g_p = jnp.kron(eye8, g16).astype(jnp.bfloat16)
    zeros = jnp.zeros((N, D_OUT), jnp.float32)
    ef_p = edge_feats.reshape(E // 8, 128)

    h = _node_proj(node_feats, proj_W, proj_b.reshape(1, D_OUT))
    for step in range(2):
        if step == 0:
            hs = _gather(h, src3)
        else:
            hs = _combine_gather(parts, conv_b.reshape(1, D_OUT), src3)
        hp = hs.reshape(E // 8, 128)
        m = _edge_messages(ef_p, hp, w1_p, b1_p, w2_p, b2_p, q_p, g_p)
        parts = _scatter(m.reshape(E, D_OUT), dst3, zeros)
    return _final(parts, conv_b.reshape(1, D_OUT),
                  pred_W, pred_b.reshape(1, 2))


# speedup vs baseline: 7.3785x; 1.0013x over previous
import jax
import jax.numpy as jnp
from jax import lax
from jax.experimental import pallas as pl
from jax.experimental.pallas import tpu as pltpu
from jax.experimental.pallas import tpu_sc as plsc

N = 10000
E = 320000
D_IN = 128
D_EDGE = 16
D_OUT = 16
D_HID = 32
D_OO = D_OUT * D_OUT

NC = 2
NS = 16
NW = NC * NS
EW = E // NW
CG = 80
NCH = EW // CG
NPT = N // NS
SUP = 25
NSUP = NCH // SUP
SB = SUP * CG

_mesh = plsc.VectorSubcoreMesh(core_axis_name="c", subcore_axis_name="s")
_sc_params = pltpu.CompilerParams(use_tc_tiling_on_sc=False)


def _proj_body(nf_ref, w_ref, b_ref, o_ref):
    x = jnp.dot(nf_ref[...], w_ref[...], preferred_element_type=jnp.float32)
    o_ref[...] = jnp.maximum(x + b_ref[...], 0.0)


def _node_proj(nf, w, b2d):
    bm = 1000
    return pl.pallas_call(
        _proj_body,
        grid=(N // bm,),
        in_specs=[
            pl.BlockSpec((bm, D_IN), lambda i: (i, 0)),
            pl.BlockSpec((D_IN, D_OUT), lambda i: (0, 0)),
            pl.BlockSpec((1, D_OUT), lambda i: (0, 0)),
        ],
        out_specs=pl.BlockSpec((bm, D_OUT), lambda i: (i, 0)),
        out_shape=jax.ShapeDtypeStruct((N, D_OUT), jnp.float32),
    )(nf, w, b2d)


def _msg_body(ef_ref, hp_ref, w1_ref, b1_ref, w2p_ref, b2p_ref, q_ref, g_ref,
              o_ref):
    a = jnp.dot(ef_ref[...].astype(jnp.bfloat16), w1_ref[...],
                preferred_element_type=jnp.float32)
    a = jnp.maximum(a + b1_ref[...], 0.0).astype(jnp.bfloat16)
    we = jnp.dot(a, w2p_ref[...], preferred_element_type=jnp.float32)
    we = we + b2p_ref[...]
    hpt = jnp.dot(hp_ref[...].astype(jnp.bfloat16), q_ref[...],
                  preferred_element_type=jnp.float32)
    prod = (we * hpt).astype(jnp.bfloat16)
    o_ref[...] = jnp.dot(prod, g_ref[...],
                         preferred_element_type=jnp.float32)


def _edge_messages(ef, hp, w1_p, b1_p, w2_p, b2_p, q_p, g_p):
    bm = 1600
    ep8 = E // 8
    return pl.pallas_call(
        _msg_body,
        grid=(ep8 // bm,),
        in_specs=[
            pl.BlockSpec((bm, 128), lambda i: (i, 0)),
            pl.BlockSpec((bm, 128), lambda i: (i, 0)),
            pl.BlockSpec((128, 8 * D_HID), lambda i: (0, 0)),
            pl.BlockSpec((1, 8 * D_HID), lambda i: (0, 0)),
            pl.BlockSpec((8 * D_HID, 8 * D_OO), lambda i: (0, 0)),
            pl.BlockSpec((1, 8 * D_OO), lambda i: (0, 0)),
            pl.BlockSpec((128, 8 * D_OO), lambda i: (0, 0)),
            pl.BlockSpec((8 * D_OO, 128), lambda i: (0, 0)),
        ],
        out_specs=pl.BlockSpec((bm, 128), lambda i: (i, 0)),
        out_shape=jax.ShapeDtypeStruct((ep8, 128), jnp.float32),
    )(ef, hp, w1_p, b1_p, w2_p, b2_p, q_p, g_p)


def _final_body(p_ref, b_ref, pw_ref, pb_ref, o_ref):
    h = jnp.maximum(p_ref[0] + p_ref[1] + b_ref[...], 0.0)
    gm = jnp.sum(h, axis=0, keepdims=True) * (1.0 / N)
    o_ref[...] = jnp.dot(gm, pw_ref[...], preferred_element_type=jnp.float32) + pb_ref[...]


def _final(parts, b2d, pw, pb2d):
    return pl.pallas_call(
        _final_body,
        in_specs=[
            pl.BlockSpec((NC, N, D_OUT), lambda: (0, 0, 0)),
            pl.BlockSpec((1, D_OUT), lambda: (0, 0)),
            pl.BlockSpec((D_OUT, 2), lambda: (0, 0)),
            pl.BlockSpec((1, 2), lambda: (0, 0)),
        ],
        out_specs=pl.BlockSpec((1, 2), lambda: (0, 0)),
        out_shape=jax.ShapeDtypeStruct((1, 2), jnp.float32),
    )(parts, b2d, pw, pb2d)


def _gather_body(h_hbm, src_hbm, out_hbm, idx_v, rows_v, sem, sem2):
    cid = lax.axis_index("c")
    sid = lax.axis_index("s")
    w = cid * NS + sid
    pltpu.sync_copy(src_hbm.at[w], idx_v)
    base = w * EW

    def fire(s, b):
        return [
            pltpu.async_copy(
                h_hbm.at[idx_v.at[s * SUP + c]],
                rows_v.at[b, pl.ds(c * CG, CG)], sem)
            for c in range(SUP)
        ]

    outs = []
    descs = fire(0, 0)
    for s in range(NSUP):
        for d in descs:
            d.wait()
        if outs:
            outs.pop(0).wait()
        if s + 1 < NSUP:
            descs = fire(s + 1, (s + 1) % 2)
        outs.append(pltpu.async_copy(
            rows_v.at[s % 2], out_hbm.at[pl.ds(base + s * SB, SB)], sem2))
    outs.pop(0).wait()


def _gather(h, src3):
    return pl.kernel(
        _gather_body,
        out_type=jax.ShapeDtypeStruct((E, D_OUT), jnp.float32),
        mesh=_mesh,
        scratch_types=[
            pltpu.VMEM((NCH, CG), jnp.int32),
            pltpu.VMEM((2, SB, D_OUT), jnp.float32),
            pltpu.SemaphoreType.DMA,
            pltpu.SemaphoreType.DMA,
        ],
        compiler_params=_sc_params,
    )(h, src3)


def _cgather_body(p_hbm, b_hbm, src_hbm, out_hbm, h_sh, idx_v, rows_v, pv0,
                  pv1, hbuf, bv, sem, sem2):
    cid = lax.axis_index("c")
    sid = lax.axis_index("s")
    w = cid * NS + sid
    nbase = sid * NPT
    pltpu.sync_copy(p_hbm.at[0].at[pl.ds(nbase, NPT)], pv0)
    pltpu.sync_copy(p_hbm.at[1].at[pl.ds(nbase, NPT)], pv1)
    pltpu.sync_copy(b_hbm, bv)
    pltpu.sync_copy(src_hbm.at[w], idx_v)

    def crow(r, carry):
        hbuf[r, :] = jnp.maximum(pv0[r, :] + pv1[r, :] + bv[0, :], 0.0)
        return carry

    lax.fori_loop(0, NPT, crow, 0)
    pltpu.sync_copy(hbuf, h_sh.at[pl.ds(nbase, NPT)])
    plsc.subcore_barrier()
    base = w * EW

    def fire(s, b):
        return [
            pltpu.async_copy(
                h_sh.at[idx_v.at[s * SUP + c]],
                rows_v.at[b, pl.ds(c * CG, CG)], sem)
            for c in range(SUP)
        ]

    outs = []
    descs = fire(0, 0)
    for s in range(NSUP):
        for d in descs:
            d.wait()
        if outs:
            outs.pop(0).wait()
        if s + 1 < NSUP:
            descs = fire(s + 1, (s + 1) % 2)
        outs.append(pltpu.async_copy(
            rows_v.at[s % 2], out_hbm.at[pl.ds(base + s * SB, SB)], sem2))
    outs.pop(0).wait()


def _combine_gather(parts, b2d, src3):
    return pl.kernel(
        _cgather_body,
        out_type=jax.ShapeDtypeStruct((E, D_OUT), jnp.float32),
        mesh=_mesh,
        scratch_types=[
            pltpu.VMEM_SHARED((N, D_OUT), jnp.float32),
            pltpu.VMEM((NCH, CG), jnp.int32),
            pltpu.VMEM((2, SB, D_OUT), jnp.float32),
            pltpu.VMEM((NPT, D_OUT), jnp.float32),
            pltpu.VMEM((NPT, D_OUT), jnp.float32),
            pltpu.VMEM((NPT, D_OUT), jnp.float32),
            pltpu.VMEM((1, D_OUT), jnp.float32),
            pltpu.SemaphoreType.DMA,
            pltpu.SemaphoreType.DMA,
        ],
        compiler_params=_sc_params,
    )(parts, b2d, src3)


def _scatter_body(m_hbm, dst_hbm, z_hbm, out_hbm, agg_sh, idx_v, m_v, row_v,
                  seml, sems):
    cid = lax.axis_index("c")
    sid = lax.axis_index("s")
    w = cid * NS + sid
    pltpu.sync_copy(z_hbm.at[pl.ds(sid * NPT, NPT)], row_v)
    pltpu.sync_copy(row_v, agg_sh.at[pl.ds(sid * NPT, NPT)])
    pltpu.sync_copy(dst_hbm.at[w], idx_v)
    plsc.subcore_barrier()
    base = w * EW

    def load(s):
        return pltpu.async_copy(
            m_hbm.at[pl.ds(base + s * SB, SB)], m_v.at[s % 2], seml)

    ld = load(0)
    prev = []
    for s in range(NSUP):
        ld.wait()
        for d in prev:
            d.wait()
        if s + 1 < NSUP:
            ld = load(s + 1)
        prev = [
            pltpu.async_copy(
                m_v.at[s % 2, pl.ds(c * CG, CG)],
                agg_sh.at[idx_v.at[s * SUP + c]], sems, add=True)
            for c in range(SUP)
        ]
    for d in prev:
        d.wait()
    plsc.subcore_barrier()
    pltpu.sync_copy(agg_sh.at[pl.ds(sid * NPT, NPT)], row_v)
    pltpu.sync_copy(row_v, out_hbm.at[cid].at[pl.ds(sid * NPT, NPT)])


def _scatter(m, dst3, zeros):
    return pl.kernel(
        _scatter_body,
        out_type=jax.ShapeDtypeStruct((NC, N, D_OUT), jnp.float32),
        mesh=_mesh,
        scratch_types=[
            pltpu.VMEM_SHARED((N, D_OUT), jnp.float32),
            pltpu.VMEM((NCH, CG), jnp.int32),
            pltpu.VMEM((2, SB, D_OUT), jnp.float32),
            pltpu.VMEM((NPT, D_OUT), jnp.float32),
            pltpu.SemaphoreType.DMA,
            pltpu.SemaphoreType.DMA,
        ],
        compiler_params=_sc_params,
    )(m, dst3, zeros)


def kernel(node_feats, edge_feats, edge_index, proj_W, proj_b, e1_W, e1_b,
           e2_W, e2_b, conv_b, pred_W, pred_b):
    src3 = edge_index[0].astype(jnp.int32).reshape(NW, NCH, CG)
    dst3 = edge_index[1].astype(jnp.int32).reshape(NW, NCH, CG)

    j = jnp.arange(D_OO)
    perm = (j % D_OUT) * D_OUT + j // D_OUT
    w2perm = e2_W[:, perm]
    b2perm = e2_b[perm]
    eye8 = jnp.eye(8, dtype=jnp.float32)
    w1_p = jnp.kron(eye8, e1_W).astype(jnp.bfloat16)
    b1_p = jnp.tile(e1_b, 8).reshape(1, 8 * D_HID)
    w2_p = jnp.kron(eye8, w2perm).astype(jnp.bfloat16)
    b2_p = jnp.tile(b2perm, 8).reshape(1, 8 * D_OO)
    q16 = jnp.concatenate([jnp.eye(D_OUT, dtype=jnp.float32)] * D_OUT, axis=1)
    q_p = jnp.kron(eye8, q16).astype(jnp.bfloat16)
    g16 = jnp.repeat(jnp.eye(D_OUT, dtype=jnp.float32), D_OUT, axis=0)
    g_p = jnp.kron(eye8, g16).astype(jnp.bfloat16)
    zeros = jnp.zeros((N, D_OUT), jnp.float32)
    ef_p = edge_feats.reshape(E // 8, 128)

    h = _node_proj(node_feats, proj_W, proj_b.reshape(1, D_OUT))
    for step in range(2):
        if step == 0:
            hs = _gather(h, src3)
        else:
            hs = _combine_gather(parts, conv_b.reshape(1, D_OUT), src3)
        hp = hs.reshape(E // 8, 128)
        m = _edge_messages(ef_p, hp, w1_p, b1_p, w2_p, b2_p, q_p, g_p)
        parts = _scatter(m.reshape(E, D_OUT), dst3, zeros)
    return _final(parts, conv_b.reshape(1, D_OUT),
                  pred_W, pred_b.reshape(1, 2))
